# baseline (speedup 1.0000x reference)
import functools

import jax
import jax.numpy as jnp
from jax import lax
from jax.experimental import pallas as pl
from jax.experimental.pallas import tpu as pltpu
from jax.experimental.pallas import tpu_sc as plsc

D_MODEL = 64
VOCAB_ROWS = 1000000
SCALE = 8.0

NC, NS = 2, 16
NW = NC * NS
N_TOK = 4096
N_POS = 200
L = 128
NBLK = N_TOK // L
PAD = 136


def _emb_body(x_hbm, tab_hbm, out_hbm, idx_v, rows_v, obuf,
              gsem0, gsem1, gsem2, osem0, osem1, osem2):
    gsem = (gsem0, gsem1, gsem2)
    osem = (osem0, osem1, osem2)
    wid = lax.axis_index("s") * NC + lax.axis_index("c")

    pltpu.sync_copy(x_hbm.at[:, pl.ds(wid * L, L)], idx_v)

    lane = lax.iota(jnp.int32, 16)
    a_idx = [(lane + 16 * c) >> 3 for c in range(4)]
    s_idx = [(lane + 16 * c) & 7 for c in range(4)]

    def fetch(j, b):
        pltpu.async_copy(tab_hbm.at[idx_v.at[j]], rows_v.at[b], gsem[b])

    def drain_g(b):
        pltpu.make_async_copy(tab_hbm.at[pl.ds(0, L)], rows_v.at[b],
                              gsem[b]).wait()

    def drain_o(b):
        pltpu.make_async_copy(out_hbm.at[0, :, 0],
                              obuf.at[b, :, :, pl.ds(0, L)], osem[b]).wait()

    def transpose_scale(b):
        @plsc.parallel_loop(0, L, unroll=8)
        def _r(r):
            l_idx = jnp.full((16,), r, dtype=jnp.int32)
            for c in range(4):
                v = rows_v[b, r, pl.ds(16 * c, 16)] * SCALE
                plsc.store_scatter(obuf.at[b], [a_idx[c], s_idx[c], l_idx], v)

    def out_copy(j, b):
        pltpu.async_copy(obuf.at[b, :, :, pl.ds(0, L)],
                         out_hbm.at[j, :, wid], osem[b])

    fetch(0, 0)
    fetch(1, 1)
    drain_g(0)
    transpose_scale(0)
    out_copy(0, 0)
    fetch(2, 2)
    drain_g(1)
    transpose_scale(1)
    out_copy(1, 1)

    @pl.loop(2, N_POS - 3, step=3)
    def _steady(jj):
        for db in range(3):
            j = jj + db
            b = (2 + db) % 3
            nb = (b + 1) % 3
            drain_o(nb)
            fetch(j + 1, nb)
            drain_g(b)
            transpose_scale(b)
            out_copy(j, b)

    drain_o(0)
    fetch(198, 0)
    drain_g(2)
    transpose_scale(2)
    out_copy(197, 2)
    drain_o(1)
    fetch(199, 1)
    drain_g(0)
    transpose_scale(0)
    out_copy(198, 0)
    drain_g(1)
    transpose_scale(1)
    out_copy(199, 1)
    for b in range(3):
        drain_o(b)


@jax.jit
def _run(x_t, tab2):
    mesh = plsc.VectorSubcoreMesh(core_axis_name="c", subcore_axis_name="s")
    f = functools.partial(
        pl.kernel,
        mesh=mesh,
        compiler_params=pltpu.CompilerParams(use_tc_tiling_on_sc=False,
                                             needs_layout_passes=False),
        out_type=jax.ShapeDtypeStruct((N_POS, D_MODEL // 8, NBLK, 8, L),
                                      jnp.float32),
        scratch_types=[
            pltpu.VMEM((N_POS, L), jnp.int32),
            pltpu.VMEM((3, L, D_MODEL), jnp.float32),
            pltpu.VMEM((3, D_MODEL // 8, 8, PAD), jnp.float32),
            pltpu.SemaphoreType.DMA,
            pltpu.SemaphoreType.DMA,
            pltpu.SemaphoreType.DMA,
            pltpu.SemaphoreType.DMA,
            pltpu.SemaphoreType.DMA,
            pltpu.SemaphoreType.DMA,
        ],
    )(_emb_body)
    return f(x_t, tab2)


def kernel(x, table):
    x_t = (x.astype(jnp.int32) * 2).T
    tab2 = jnp.pad(table, ((0, 0), (0, 64))).reshape(2 * VOCAB_ROWS, D_MODEL)
    out5 = _run(x_t, tab2)
    out = jnp.transpose(out5, (2, 4, 0, 1, 3)).reshape(N_TOK, N_POS, D_MODEL)
    return out


# speedup vs baseline: 1.0549x; 1.0549x over previous
import functools

import jax
import jax.numpy as jnp
from jax import lax
from jax.experimental import pallas as pl
from jax.experimental.pallas import tpu as pltpu
from jax.experimental.pallas import tpu_sc as plsc

D_MODEL = 64
VOCAB_ROWS = 1000000
SCALE = 8.0

NC, NS = 2, 16
NW = NC * NS
N_TOK = 4096
N_POS = 200
L = 128
NBLK = N_TOK // L
PAD = 136


def _emb_body(x_hbm, tab_hbm, out_hbm, idx_v, rows_v, obuf,
              gsem0, gsem1, gsem2, gsem3, osem0, osem1, osem2, osem3):
    gsem = (gsem0, gsem1, gsem2, gsem3)
    osem = (osem0, osem1, osem2, osem3)
    wid = lax.axis_index("s") * NC + lax.axis_index("c")

    pltpu.sync_copy(x_hbm.at[:, pl.ds(wid * L, L)], idx_v)

    lane = lax.iota(jnp.int32, 16)
    a_idx = [(lane + 16 * c) >> 3 for c in range(4)]
    s_idx = [(lane + 16 * c) & 7 for c in range(4)]

    def fetch(j, b):
        pltpu.async_copy(tab_hbm.at[idx_v.at[j]], rows_v.at[b], gsem[b])

    def drain_g(b):
        pltpu.make_async_copy(tab_hbm.at[pl.ds(0, L)], rows_v.at[b],
                              gsem[b]).wait()

    def drain_o(b):
        pltpu.make_async_copy(out_hbm.at[0, :, 0],
                              obuf.at[b, :, :, pl.ds(0, L)], osem[b]).wait()

    def transpose_scale(b):
        @plsc.parallel_loop(0, L, unroll=4)
        def _r(r):
            l_idx = jnp.full((16,), r, dtype=jnp.int32)
            for c in range(4):
                v = rows_v[b, r, pl.ds(16 * c, 16)] * SCALE
                plsc.store_scatter(obuf.at[b], [a_idx[c], s_idx[c], l_idx], v)

    def out_copy(j, b):
        pltpu.async_copy(obuf.at[b, :, :, pl.ds(0, L)],
                         out_hbm.at[j, :, wid], osem[b])

    fetch(0, 0)
    fetch(1, 1)
    fetch(2, 2)
    drain_g(0)
    transpose_scale(0)
    out_copy(0, 0)
    fetch(3, 3)
    drain_g(1)
    transpose_scale(1)
    out_copy(1, 1)

    @pl.loop(2, N_POS - 2, step=4)
    def _steady(jj):
        for db in range(4):
            j = jj + db
            b = (2 + db) % 4
            nb = (b + 2) % 4
            drain_o(nb)
            fetch(j + 2, nb)
            drain_g(b)
            transpose_scale(b)
            out_copy(j, b)

    drain_o(0)
    drain_g(2)
    transpose_scale(2)
    out_copy(198, 2)
    drain_o(1)
    drain_g(3)
    transpose_scale(3)
    out_copy(199, 3)
    drain_o(2)
    drain_o(3)


@jax.jit
def _run(x_t, tab2):
    mesh = plsc.VectorSubcoreMesh(core_axis_name="c", subcore_axis_name="s")
    f = functools.partial(
        pl.kernel,
        mesh=mesh,
        compiler_params=pltpu.CompilerParams(use_tc_tiling_on_sc=False,
                                             needs_layout_passes=False),
        out_type=jax.ShapeDtypeStruct((N_POS, D_MODEL // 8, NBLK, 8, L),
                                      jnp.float32),
        scratch_types=[
            pltpu.VMEM((N_POS, L), jnp.int32),
            pltpu.VMEM((4, L, D_MODEL), jnp.float32),
            pltpu.VMEM((4, D_MODEL // 8, 8, PAD), jnp.float32),
            pltpu.SemaphoreType.DMA,
            pltpu.SemaphoreType.DMA,
            pltpu.SemaphoreType.DMA,
            pltpu.SemaphoreType.DMA,
            pltpu.SemaphoreType.DMA,
            pltpu.SemaphoreType.DMA,
            pltpu.SemaphoreType.DMA,
            pltpu.SemaphoreType.DMA,
        ],
    )(_emb_body)
    return f(x_t, tab2)


def kernel(x, table):
    x_t = (x.astype(jnp.int32) * 2).T
    tab2 = jnp.pad(table, ((0, 0), (0, 64))).reshape(2 * VOCAB_ROWS, D_MODEL)
    out5 = _run(x_t, tab2)
    out = jnp.transpose(out5, (2, 4, 0, 1, 3)).reshape(N_TOK, N_POS, D_MODEL)
    return out
